# Initial kernel scaffold; baseline (speedup 1.0000x reference)
import jax
import jax.numpy as jnp
from jax.experimental import pallas as pl


def kernel(x, edge_index, W_conv1, b_conv1, W_conv2, b_conv2, W_read, b_read):
    raise NotImplementedError("write your pallas kernel here")


# speedup vs baseline: 10.2379x; 10.2379x over previous
import functools

import jax
import jax.numpy as jnp
from jax import lax
from jax.experimental import pallas as pl
from jax.experimental.pallas import tpu as pltpu
from jax.experimental.pallas import tpu_sc as plsc

NC = 2
NS = 16
NW = NC * NS
K = 128
R = 256


def _make_deg_kernel(n_pad, ept):
    mesh = plsc.VectorSubcoreMesh(
        core_axis_name="c", subcore_axis_name="s",
        num_cores=NC, num_subcores=NS)

    @functools.partial(
        pl.kernel,
        out_type=jax.ShapeDtypeStruct((NW, n_pad), jnp.float32),
        mesh=mesh,
        compiler_params=pltpu.CompilerParams(needs_layout_passes=False),
        scratch_types=[
            pltpu.VMEM((ept,), jnp.int32),
            pltpu.VMEM((n_pad,), jnp.float32),
            pltpu.VMEM((32,), jnp.float32),
        ],
    )
    def deg_kernel(dst_hbm, const_hbm, deg_hbm, idx_v, hist_v, const_v):
        c = lax.axis_index("c")
        s = lax.axis_index("s")
        wid = s * NC + c
        pltpu.sync_copy(dst_hbm.at[wid], idx_v)
        pltpu.sync_copy(const_hbm, const_v)
        zv = const_v[pl.ds(16, 16)]
        ov = const_v[pl.ds(0, 16)]

        def zero_body(i, carry):
            hist_v[pl.ds(i * 16, 16)] = zv
            return carry

        lax.fori_loop(0, n_pad // 16, zero_body, 0)

        def add_body(i, carry):
            idx = idx_v[pl.ds(i * 16, 16)]
            plsc.addupdate_scatter(hist_v, [idx], ov)
            return carry

        lax.fori_loop(0, ept // 16, add_body, 0)
        pltpu.sync_copy(hist_v, deg_hbm.at[wid])

    return deg_kernel


def _make_scatter_kernel(n_pad, ch, d):
    mesh = plsc.VectorSubcoreMesh(
        core_axis_name="c", subcore_axis_name="s",
        num_cores=NC, num_subcores=NS)
    rows_t = n_pad // NS

    @functools.partial(
        pl.kernel,
        out_type=jax.ShapeDtypeStruct((NC, n_pad, d), jnp.float32),
        mesh=mesh,
        scratch_types=[
            pltpu.VMEM((ch, K), jnp.int32),
            pltpu.VMEM((ch, K), jnp.int32),
            pltpu.VMEM((K, d), jnp.float32),
            pltpu.VMEM_SHARED((n_pad, d), jnp.float32),
            pltpu.SemaphoreType.DMA,
        ],
    )
    def scatter_kernel(y_hbm, src_hbm, dst_hbm, zeros_hbm, z_hbm,
                       src_v, dst_v, buf, z_sp, gsem):
        c = lax.axis_index("c")
        s = lax.axis_index("s")
        wid = s * NC + c
        pltpu.sync_copy(src_hbm.at[wid], src_v)
        pltpu.sync_copy(dst_hbm.at[wid], dst_v)
        pltpu.sync_copy(zeros_hbm.at[pl.ds(s * rows_t, rows_t)],
                        z_sp.at[pl.ds(s * rows_t, rows_t)])
        plsc.subcore_barrier()
        for j in range(ch):
            pltpu.async_copy(y_hbm.at[src_v.at[j]], buf, gsem).wait()
            pltpu.sync_copy(buf, z_sp.at[dst_v.at[j]], add=True)
        plsc.subcore_barrier()
        pltpu.sync_copy(z_sp.at[pl.ds(s * rows_t, rows_t)],
                        z_hbm.at[c, pl.ds(s * rows_t, rows_t)])

    return scatter_kernel


def _scale_body(deg_ref, x_ref, y_ref, dinv_ref):
    deg = jnp.sum(deg_ref[...], axis=1, keepdims=True) + 1.0
    dinv = lax.rsqrt(deg)
    dinvb = jnp.broadcast_to(dinv, x_ref.shape)
    dinv_ref[...] = dinvb
    y_ref[...] = x_ref[...] * dinvb


def _conv_body(z_ref, dinv_ref, w_ref, b_ref, y2_ref):
    dinvb = dinv_ref[...]
    agg = (z_ref[0] + z_ref[1]) * dinvb
    h = jnp.tanh(jnp.dot(agg, w_ref[...],
                         preferred_element_type=jnp.float32) + b_ref[...])
    y2_ref[...] = h * dinvb


def _final_body(z_ref, dinv_ref, w_ref, b_ref, wr_ref, br_ref, out_ref):
    agg = (z_ref[0] + z_ref[1]) * dinv_ref[...]
    h = jnp.tanh(jnp.dot(agg, w_ref[...],
                         preferred_element_type=jnp.float32) + b_ref[...])
    out_ref[...] = jnp.dot(h, wr_ref[...],
                           preferred_element_type=jnp.float32) + br_ref[...]


def _tc_scale(degp, x_pad, n_pad, d):
    grid = (n_pad // R,)
    return pl.pallas_call(
        _scale_body,
        grid=grid,
        in_specs=[
            pl.BlockSpec((R, NW), lambda i: (i, 0)),
            pl.BlockSpec((R, d), lambda i: (i, 0)),
        ],
        out_specs=[
            pl.BlockSpec((R, d), lambda i: (i, 0)),
            pl.BlockSpec((R, d), lambda i: (i, 0)),
        ],
        out_shape=[
            jax.ShapeDtypeStruct((n_pad, d), jnp.float32),
            jax.ShapeDtypeStruct((n_pad, d), jnp.float32),
        ],
    )(degp, x_pad)


def _tc_conv(zp, dinvb, w, b, n_pad, d):
    grid = (n_pad // R,)
    return pl.pallas_call(
        _conv_body,
        grid=grid,
        in_specs=[
            pl.BlockSpec((NC, R, d), lambda i: (0, i, 0)),
            pl.BlockSpec((R, d), lambda i: (i, 0)),
            pl.BlockSpec((d, d), lambda i: (0, 0)),
            pl.BlockSpec((1, d), lambda i: (0, 0)),
        ],
        out_specs=pl.BlockSpec((R, d), lambda i: (i, 0)),
        out_shape=jax.ShapeDtypeStruct((n_pad, d), jnp.float32),
    )(zp, dinvb, w, b)


def _tc_final(zp, dinvb, w, b, wr, br, n_pad, d, out_d):
    grid = (n_pad // R,)
    return pl.pallas_call(
        _final_body,
        grid=grid,
        in_specs=[
            pl.BlockSpec((NC, R, d), lambda i: (0, i, 0)),
            pl.BlockSpec((R, d), lambda i: (i, 0)),
            pl.BlockSpec((d, d), lambda i: (0, 0)),
            pl.BlockSpec((1, d), lambda i: (0, 0)),
            pl.BlockSpec((d, out_d), lambda i: (0, 0)),
            pl.BlockSpec((1, out_d), lambda i: (0, 0)),
        ],
        out_specs=pl.BlockSpec((R, out_d), lambda i: (i, 0)),
        out_shape=jax.ShapeDtypeStruct((n_pad, out_d), jnp.float32),
    )(zp, dinvb, w, b, wr, br)


def kernel(x, edge_index, W_conv1, b_conv1, W_conv2, b_conv2, W_read, b_read):
    n, d = x.shape
    out_d = W_read.shape[1]
    e = edge_index.shape[1]

    n_pad = -(-n // 2560) * 2560
    if n_pad == n:
        n_pad += 2560
    ept = -(-e // (NW * K)) * K
    e_pad = ept * NW
    ch = ept // K

    ei = edge_index.astype(jnp.int32)
    src = jnp.concatenate(
        [ei[0], jnp.zeros((e_pad - e,), jnp.int32)]).reshape(NW, ch, K)
    dst = jnp.concatenate(
        [ei[1], jnp.full((e_pad - e,), n, jnp.int32)]).reshape(NW, ch, K)

    x_pad = jnp.pad(x, ((0, n_pad - n), (0, 0)))
    zeros_d = jnp.zeros((n_pad, d), jnp.float32)

    const32 = jnp.concatenate(
        [jnp.ones((16,), jnp.float32), jnp.zeros((16,), jnp.float32)])
    hists = _make_deg_kernel(n_pad, ept)(dst.reshape(NW, ept), const32)
    y1, dinvb = _tc_scale(hists.T, x_pad, n_pad, d)

    sc_scatter = _make_scatter_kernel(n_pad, ch, d)
    z1 = sc_scatter(y1, src, dst, zeros_d)
    y2 = _tc_conv(z1, dinvb, W_conv1, b_conv1.reshape(1, d), n_pad, d)
    z2 = sc_scatter(y2, src, dst, zeros_d)
    out = _tc_final(z2, dinvb, W_conv2, b_conv2.reshape(1, d),
                    W_read, b_read.reshape(1, out_d), n_pad, d, out_d)
    return out[:n]


# speedup vs baseline: 11.1461x; 1.0887x over previous
import functools

import jax
import jax.numpy as jnp
from jax import lax
from jax.experimental import pallas as pl
from jax.experimental.pallas import tpu as pltpu
from jax.experimental.pallas import tpu_sc as plsc

NC = 2
NS = 16
NW = NC * NS
K = 128
NB = 2
R = 256


def _make_deg_kernel(n_pad, ept):
    mesh = plsc.VectorSubcoreMesh(
        core_axis_name="c", subcore_axis_name="s",
        num_cores=NC, num_subcores=NS)

    @functools.partial(
        pl.kernel,
        out_type=jax.ShapeDtypeStruct((NW, n_pad), jnp.float32),
        mesh=mesh,
        compiler_params=pltpu.CompilerParams(needs_layout_passes=False),
        scratch_types=[
            pltpu.VMEM((ept,), jnp.int32),
            pltpu.VMEM((n_pad,), jnp.float32),
            pltpu.VMEM((32,), jnp.float32),
        ],
    )
    def deg_kernel(dst_hbm, const_hbm, deg_hbm, idx_v, hist_v, const_v):
        c = lax.axis_index("c")
        s = lax.axis_index("s")
        wid = s * NC + c
        pltpu.sync_copy(dst_hbm.at[wid], idx_v)
        pltpu.sync_copy(const_hbm, const_v)
        zv = const_v[pl.ds(16, 16)]
        ov = const_v[pl.ds(0, 16)]

        def zero_body(i, carry):
            hist_v[pl.ds(i * 16, 16)] = zv
            return carry

        lax.fori_loop(0, n_pad // 16, zero_body, 0)

        def add_body(i, carry):
            idx = idx_v[pl.ds(i * 16, 16)]
            plsc.addupdate_scatter(hist_v, [idx], ov)
            return carry

        lax.fori_loop(0, ept // 16, add_body, 0)
        pltpu.sync_copy(hist_v, deg_hbm.at[wid])

    return deg_kernel


def _make_scatter_kernel(n_pad, n_sp, ch, d):
    mesh = plsc.VectorSubcoreMesh(
        core_axis_name="c", subcore_axis_name="s",
        num_cores=NC, num_subcores=NS)
    rows_t = n_sp // NS

    @functools.partial(
        pl.kernel,
        out_type=jax.ShapeDtypeStruct((NC, n_pad, d), jnp.float32),
        mesh=mesh,
        scratch_types=[
            pltpu.VMEM((ch, K), jnp.int32),
            pltpu.VMEM((NB, K), jnp.int32),
            [pltpu.VMEM((K, d), jnp.float32)] * NB,
            pltpu.VMEM_SHARED((n_sp, d), jnp.float32),
            [pltpu.SemaphoreType.DMA] * NB,
            [pltpu.SemaphoreType.DMA] * NB,
        ],
    )
    def scatter_kernel(y_hbm, src_hbm, dst_hbm, zeros_hbm, z_hbm,
                       src_v, dring, bufs, z_sp, gsems, dsems):
        c = lax.axis_index("c")
        s = lax.axis_index("s")
        wid = s * NC + c
        pltpu.sync_copy(src_hbm.at[wid], src_v)
        pltpu.sync_copy(zeros_hbm.at[pl.ds(s * rows_t, rows_t)],
                        z_sp.at[pl.ds(s * rows_t, rows_t)])
        plsc.subcore_barrier()

        def gather(j):
            return pltpu.async_copy(
                y_hbm.at[src_v.at[j]], bufs[j % NB], gsems[j % NB])

        def dload(j):
            return pltpu.async_copy(
                dst_hbm.at[wid * ch + j], dring.at[j % NB], dsems[j % NB])

        gd = {j: gather(j) for j in range(min(NB - 1, ch))}
        dd = {j: dload(j) for j in range(min(NB - 1, ch))}
        for j in range(ch):
            p = j % NB
            gd[j].wait()
            dd[j].wait()
            if j + NB - 1 < ch:
                gd[j + NB - 1] = gather(j + NB - 1)
                dd[j + NB - 1] = dload(j + NB - 1)
            pltpu.sync_copy(bufs[p], z_sp.at[dring.at[p]], add=True)
        plsc.subcore_barrier()
        pltpu.sync_copy(z_sp.at[pl.ds(s * rows_t, rows_t)],
                        z_hbm.at[c, pl.ds(s * rows_t, rows_t)])

    return scatter_kernel


def _scale_body(deg_ref, x_ref, y_ref, dinv_ref):
    deg = jnp.sum(deg_ref[...], axis=1, keepdims=True) + 1.0
    dinv = lax.rsqrt(deg)
    dinvb = jnp.broadcast_to(dinv, x_ref.shape)
    dinv_ref[...] = dinvb
    y_ref[...] = x_ref[...] * dinvb


def _conv_body(z_ref, dinv_ref, w_ref, b_ref, y2_ref):
    dinvb = dinv_ref[...]
    agg = (z_ref[0] + z_ref[1]) * dinvb
    h = jnp.tanh(jnp.dot(agg, w_ref[...],
                         preferred_element_type=jnp.float32) + b_ref[...])
    y2_ref[...] = h * dinvb


def _final_body(z_ref, dinv_ref, w_ref, b_ref, wr_ref, br_ref, out_ref):
    agg = (z_ref[0] + z_ref[1]) * dinv_ref[...]
    h = jnp.tanh(jnp.dot(agg, w_ref[...],
                         preferred_element_type=jnp.float32) + b_ref[...])
    out_ref[...] = jnp.dot(h, wr_ref[...],
                           preferred_element_type=jnp.float32) + br_ref[...]


def _tc_scale(degp, x_pad, n_pad, d):
    grid = (n_pad // R,)
    return pl.pallas_call(
        _scale_body,
        grid=grid,
        in_specs=[
            pl.BlockSpec((R, NW), lambda i: (i, 0)),
            pl.BlockSpec((R, d), lambda i: (i, 0)),
        ],
        out_specs=[
            pl.BlockSpec((R, d), lambda i: (i, 0)),
            pl.BlockSpec((R, d), lambda i: (i, 0)),
        ],
        out_shape=[
            jax.ShapeDtypeStruct((n_pad, d), jnp.float32),
            jax.ShapeDtypeStruct((n_pad, d), jnp.float32),
        ],
    )(degp, x_pad)


def _tc_conv(zp, dinvb, w, b, n_pad, d):
    grid = (n_pad // R,)
    return pl.pallas_call(
        _conv_body,
        grid=grid,
        in_specs=[
            pl.BlockSpec((NC, R, d), lambda i: (0, i, 0)),
            pl.BlockSpec((R, d), lambda i: (i, 0)),
            pl.BlockSpec((d, d), lambda i: (0, 0)),
            pl.BlockSpec((1, d), lambda i: (0, 0)),
        ],
        out_specs=pl.BlockSpec((R, d), lambda i: (i, 0)),
        out_shape=jax.ShapeDtypeStruct((n_pad, d), jnp.float32),
    )(zp, dinvb, w, b)


def _tc_final(zp, dinvb, w, b, wr, br, n_pad, d, out_d):
    grid = (n_pad // R,)
    return pl.pallas_call(
        _final_body,
        grid=grid,
        in_specs=[
            pl.BlockSpec((NC, R, d), lambda i: (0, i, 0)),
            pl.BlockSpec((R, d), lambda i: (i, 0)),
            pl.BlockSpec((d, d), lambda i: (0, 0)),
            pl.BlockSpec((1, d), lambda i: (0, 0)),
            pl.BlockSpec((d, out_d), lambda i: (0, 0)),
            pl.BlockSpec((1, out_d), lambda i: (0, 0)),
        ],
        out_specs=pl.BlockSpec((R, out_d), lambda i: (i, 0)),
        out_shape=jax.ShapeDtypeStruct((n_pad, out_d), jnp.float32),
    )(zp, dinvb, w, b, wr, br)


def kernel(x, edge_index, W_conv1, b_conv1, W_conv2, b_conv2, W_read, b_read):
    n, d = x.shape
    out_d = W_read.shape[1]
    e = edge_index.shape[1]

    n_pad = -(-n // 2560) * 2560
    if n_pad == n:
        n_pad += 2560
    n_sp = -(-(n + 1) // 128) * 128
    ept = -(-e // (NW * K)) * K
    e_pad = ept * NW
    ch = ept // K

    ei = edge_index.astype(jnp.int32)
    src = jnp.concatenate(
        [ei[0], jnp.zeros((e_pad - e,), jnp.int32)]).reshape(NW, ch, K)
    dst = jnp.concatenate(
        [ei[1], jnp.full((e_pad - e,), n, jnp.int32)]).reshape(NW, ch, K)

    x_pad = jnp.pad(x, ((0, n_pad - n), (0, 0)))
    zeros_d = jnp.zeros((n_pad, d), jnp.float32)

    const32 = jnp.concatenate(
        [jnp.ones((16,), jnp.float32), jnp.zeros((16,), jnp.float32)])
    hists = _make_deg_kernel(n_pad, ept)(dst.reshape(NW, ept), const32)
    y1, dinvb = _tc_scale(hists.T, x_pad, n_pad, d)

    sc_scatter = _make_scatter_kernel(n_pad, n_sp, ch, d)
    z1 = sc_scatter(y1, src, dst.reshape(NW * ch, K), zeros_d)
    y2 = _tc_conv(z1, dinvb, W_conv1, b_conv1.reshape(1, d), n_pad, d)
    z2 = sc_scatter(y2, src, dst.reshape(NW * ch, K), zeros_d)
    out = _tc_final(z2, dinvb, W_conv2, b_conv2.reshape(1, d),
                    W_read, b_read.reshape(1, out_d), n_pad, d, out_d)
    return out[:n]
